# baseline (speedup 1.0000x reference)
import functools

import jax
import jax.numpy as jnp
from jax import lax
from jax.experimental import pallas as pl
from jax.experimental.pallas import tpu as pltpu, tpu_sc as plsc

NUM_CORES = 2
NUM_SUBCORES = 16
LANES = 16
NW = NUM_CORES * NUM_SUBCORES

BATCH = 16384
EMBED = 64
B_PER_W = BATCH // NW
CHUNK = 128
NCHUNK = B_PER_W // CHUNK


def _make_kernel(num_users, num_items):
    mesh = plsc.VectorSubcoreMesh(core_axis_name="c", subcore_axis_name="s")

    @functools.partial(
        pl.kernel,
        mesh=mesh,
        out_type=jax.ShapeDtypeStruct((BATCH,), jnp.float32),
        compiler_params=pltpu.CompilerParams(use_tc_tiling_on_sc=False),
        scratch_types=[
            pltpu.VMEM((NCHUNK, CHUNK), jnp.int32),
            pltpu.VMEM((NCHUNK, CHUNK), jnp.int32),
            pltpu.VMEM((B_PER_W, EMBED), jnp.float32),
            pltpu.VMEM((B_PER_W, EMBED), jnp.float32),
            pltpu.VMEM((B_PER_W,), jnp.float32),
            pltpu.SemaphoreType.DMA,
        ],
    )
    def dot_kernel(uids_hbm, iids_hbm, utab_hbm, itab_hbm, out_hbm,
                   uidx_v, iidx_v, urows_v, irows_v, out_v, sem):
        wid = lax.axis_index("s") * NUM_CORES + lax.axis_index("c")
        base = wid * B_PER_W

        pltpu.sync_copy(uids_hbm.at[wid], uidx_v)
        pltpu.sync_copy(iids_hbm.at[wid], iidx_v)

        copies = []
        for j in range(NCHUNK):
            dst = urows_v.at[pl.ds(j * CHUNK, CHUNK)]
            copies.append(pltpu.async_copy(utab_hbm.at[uidx_v.at[j]], dst, sem))
            dst = irows_v.at[pl.ds(j * CHUNK, CHUNK)]
            copies.append(pltpu.async_copy(itab_hbm.at[iidx_v.at[j]], dst, sem))
        for c in copies:
            c.wait()

        lane_ids = lax.iota(jnp.int32, LANES)
        perms = [(lane_ids + sh) % LANES for sh in (8, 4, 2, 1)]
        dnums = lax.GatherDimensionNumbers(
            offset_dims=(), collapsed_slice_dims=(0,), start_index_map=(0,))

        def lane_sum(x):
            for perm in perms:
                rot = lax.gather(
                    x, perm[:, None], dnums, (1,),
                    mode=lax.GatherScatterMode.PROMISE_IN_BOUNDS)
                x = x + rot
            return x

        def group_body(g, carry):
            def row_body(j, acc):
                b = g * LANES + j
                s = None
                for q in range(EMBED // LANES):
                    u = urows_v[b, pl.ds(q * LANES, LANES)]
                    v = irows_v[b, pl.ds(q * LANES, LANES)]
                    p = u * v
                    s = p if s is None else s + p
                dot = lane_sum(s)
                return jnp.where(lane_ids == j, dot, acc)

            acc = lax.fori_loop(0, LANES, row_body,
                                jnp.zeros((LANES,), jnp.float32))
            out_v[pl.ds(g * LANES, LANES)] = acc
            return carry

        lax.fori_loop(0, B_PER_W // LANES, group_body, 0)

        pltpu.sync_copy(out_v, out_hbm.at[pl.ds(base, B_PER_W)])

    return dot_kernel


@jax.jit
def kernel(user_ids, item_ids, user_table, item_table):
    uids = user_ids.astype(jnp.int32).reshape(NW, NCHUNK, CHUNK)
    iids = item_ids.astype(jnp.int32).reshape(NW, NCHUNK, CHUNK)
    fn = _make_kernel(user_table.shape[0], item_table.shape[0])
    return fn(uids, iids, user_table, item_table)
